# baseline (speedup 1.0000x reference)
import jax
import jax.numpy as jnp
from jax.experimental import pallas as pl
from jax.experimental.pallas import tpu as pltpu

B, N, C = 16, 20000, 80
K = 300
LANES = 128
ROWS = (N * C) // LANES
GROUPS = 98
RPAD = GROUPS * LANES
NEG = -3.0e38
BGRP = 4
CH = 2000


def _select_kernel(flat_hbm, idx_ref, sc_ref, data, rmax, sem):
    step = pl.program_id(0)

    for j in range(BGRP):
        data[j, ROWS:RPAD, :] = jnp.full((RPAD - ROWS, LANES), NEG, jnp.float32)
        cp = pltpu.make_async_copy(
            flat_hbm.at[step * BGRP + j],
            data.at[j, pl.ds(0, ROWS), :],
            sem,
        )
        cp.start()
        cp.wait()
        rmax[j] = jnp.max(data[j].reshape(GROUPS, LANES, LANES), axis=2)

    liniota = (jax.lax.broadcasted_iota(jnp.int32, (GROUPS, LANES), 0) * LANES
               + jax.lax.broadcasted_iota(jnp.int32, (GROUPS, LANES), 1))
    laneiota = jax.lax.broadcasted_iota(jnp.int32, (1, LANES), 1)
    BIG = jnp.int32(2**30)

    def body(k, _):
        for j in range(BGRP):
            gm = rmax[j]
            m = jnp.max(gm)
            r = jnp.min(jnp.where(gm == m, liniota, BIG))
            row = data[j, pl.ds(r, 1), :]
            l = jnp.min(jnp.where(row == m, laneiota, BIG))

            roww = jnp.where(laneiota == l, NEG, row)
            data[j, pl.ds(r, 1), :] = roww
            nm = jnp.max(roww)
            a = r // LANES
            s = r - a * LANES
            rrow = rmax[j, pl.ds(a, 1), :]
            rmax[j, pl.ds(a, 1), :] = jnp.where(laneiota == s, nm, rrow)

            idx_ref[j, pl.ds(k, 1), 0] = jnp.reshape(r * LANES + l, (1,))
            sc_ref[j, pl.ds(k, 1), 0] = jnp.reshape(m, (1,))
        return 0

    jax.lax.fori_loop(0, K, body, 0)
    sc_ref[...] = jax.nn.sigmoid(sc_ref[...])


def _decode_kernel(idx_ref, boxes_ref, scale_ref, lab_ref, box_ref):
    idxv = idx_ref[0]
    q = idxv // C
    lab_ref[0] = idxv - q * C

    acc = jnp.zeros((K, 4), jnp.float32)
    ci = jax.lax.broadcasted_iota(jnp.int32, (4, 4), 0)
    cj = jax.lax.broadcasted_iota(jnp.int32, (4, 4), 1)
    conv = jnp.where(
        ci % 2 == cj % 2,
        jnp.where(ci < 2, 1.0, jnp.where(cj < 2, -0.5, 0.5)),
        0.0,
    ).astype(jnp.float32)
    nch = N // CH
    for c in range(nch):
        ids = c * CH + jax.lax.broadcasted_iota(jnp.int32, (K, CH), 1)
        oh = (ids == q).astype(jnp.float32)
        bch = boxes_ref[0, c * CH:(c + 1) * CH, :]
        acc = acc + jnp.dot(oh, bch, preferred_element_type=jnp.float32)
    box_ref[0] = jnp.dot(acc, conv,
                         preferred_element_type=jnp.float32) * scale_ref[0]


def kernel(pred_logits, pred_boxes, orig_target_sizes):
    flat = pred_logits.reshape(B, ROWS, LANES)
    scale4 = jnp.tile(orig_target_sizes.astype(jnp.float32),
                      (1, 2)).reshape(B, 1, 4)

    idx, scores = pl.pallas_call(
        _select_kernel,
        grid=(B // BGRP,),
        in_specs=[pl.BlockSpec(memory_space=pl.ANY)],
        out_specs=[
            pl.BlockSpec((BGRP, K, 1), lambda i: (i, 0, 0)),
            pl.BlockSpec((BGRP, K, 1), lambda i: (i, 0, 0)),
        ],
        out_shape=[
            jax.ShapeDtypeStruct((B, K, 1), jnp.int32),
            jax.ShapeDtypeStruct((B, K, 1), jnp.float32),
        ],
        scratch_shapes=[
            pltpu.VMEM((BGRP, RPAD, LANES), jnp.float32),
            pltpu.VMEM((BGRP, GROUPS, LANES), jnp.float32),
            pltpu.SemaphoreType.DMA,
        ],
    )(flat)

    labels, boxes = pl.pallas_call(
        _decode_kernel,
        grid=(B,),
        in_specs=[
            pl.BlockSpec((1, K, 1), lambda b: (b, 0, 0)),
            pl.BlockSpec((1, N, 4), lambda b: (b, 0, 0)),
            pl.BlockSpec((1, 1, 4), lambda b: (b, 0, 0)),
        ],
        out_specs=[
            pl.BlockSpec((1, K, 1), lambda b: (b, 0, 0)),
            pl.BlockSpec((1, K, 4), lambda b: (b, 0, 0)),
        ],
        out_shape=[
            jax.ShapeDtypeStruct((B, K, 1), jnp.int32),
            jax.ShapeDtypeStruct((B, K, 4), jnp.float32),
        ],
    )(idx, pred_boxes, scale4)

    return labels[:, :, 0], boxes, scores[:, :, 0]


# speedup vs baseline: 1.1874x; 1.1874x over previous
import jax
import jax.numpy as jnp
from jax.experimental import pallas as pl
from jax.experimental.pallas import tpu as pltpu

B, N, C = 16, 20000, 80
K = 300
LANES = 128
GROUPS = 157
NPAD = GROUPS * LANES
NEG = -3.0e38
BGRP = 4
CH = 2000


def _select_kernel(flat_hbm, lab_ref, q_ref, sc_ref, *scratch):
    datas = scratch[0:BGRP]
    rmaxs = scratch[BGRP:2 * BGRP]
    sem = scratch[2 * BGRP]
    step = pl.program_id(0)

    nfull = (N // LANES) * LANES
    for j in range(BGRP):
        data = datas[j]
        cp = pltpu.make_async_copy(flat_hbm.at[step * BGRP + j], data, sem)
        cp.start()
        cp.wait()
        rmax = rmaxs[j]
        rmax[...] = jnp.full((GROUPS, LANES), NEG, jnp.float32)
        rm = jnp.max(data[...], axis=1)
        rmax[0:nfull // LANES, :] = rm[0:nfull].reshape(nfull // LANES, LANES)
        rmax[nfull // LANES, 0:N - nfull] = rm[nfull:N]

    liniota = (jax.lax.broadcasted_iota(jnp.int32, (GROUPS, LANES), 0) * LANES
               + jax.lax.broadcasted_iota(jnp.int32, (GROUPS, LANES), 1))
    laneiota = jax.lax.broadcasted_iota(jnp.int32, (1, C), 1)
    riota = jax.lax.broadcasted_iota(jnp.int32, (1, LANES), 1)
    BIG = jnp.int32(2**30)

    def body(k, _):
        for j in range(BGRP):
            data = datas[j]
            rmax = rmaxs[j]
            gm = rmax[...]
            m = jnp.max(gm)
            r = jnp.min(jnp.where(gm == m, liniota, BIG))
            row = data[pl.ds(r, 1), :]
            l = jnp.min(jnp.where(row == m, laneiota, BIG))

            roww = jnp.where(laneiota == l, NEG, row)
            data[pl.ds(r, 1), :] = roww
            nm = jnp.max(roww)
            a = r // LANES
            s = r - a * LANES
            rrow = rmax[pl.ds(a, 1), :]
            rmax[pl.ds(a, 1), :] = jnp.where(riota == s, nm, rrow)

            lab_ref[j, pl.ds(k, 1), 0] = jnp.reshape(l, (1,))
            q_ref[j, pl.ds(k, 1), 0] = jnp.reshape(r, (1,))
            sc_ref[j, pl.ds(k, 1), 0] = jnp.reshape(m, (1,))
        return 0

    jax.lax.fori_loop(0, K, body, 0)
    sc_ref[...] = jax.nn.sigmoid(sc_ref[...])


def _decode_kernel(q_ref, boxes_ref, scale_ref, box_ref):
    q = q_ref[0]

    acc = jnp.zeros((K, 4), jnp.float32)
    for c in range(N // CH):
        ids = c * CH + jax.lax.broadcasted_iota(jnp.int32, (K, CH), 1)
        oh = (ids == q).astype(jnp.float32)
        bch = boxes_ref[0, c * CH:(c + 1) * CH, :]
        acc = acc + jnp.dot(oh, bch,
                            preferred_element_type=jnp.float32,
                            precision=jax.lax.Precision.HIGHEST)
    ci = jax.lax.broadcasted_iota(jnp.int32, (4, 4), 0)
    cj = jax.lax.broadcasted_iota(jnp.int32, (4, 4), 1)
    conv = jnp.where(
        ci % 2 == cj % 2,
        jnp.where(ci < 2, 1.0, jnp.where(cj < 2, -0.5, 0.5)),
        0.0,
    ).astype(jnp.float32)
    box_ref[0] = jnp.dot(acc, conv,
                         preferred_element_type=jnp.float32,
                         precision=jax.lax.Precision.HIGHEST) * scale_ref[0]


def kernel(pred_logits, pred_boxes, orig_target_sizes):
    scale4 = jnp.tile(orig_target_sizes.astype(jnp.float32),
                      (1, 2)).reshape(B, 1, 4)

    scratches = (
        [pltpu.VMEM((N, C), jnp.float32) for _ in range(BGRP)]
        + [pltpu.VMEM((GROUPS, LANES), jnp.float32) for _ in range(BGRP)]
        + [pltpu.SemaphoreType.DMA]
    )
    labels, qidx, scores = pl.pallas_call(
        _select_kernel,
        grid=(B // BGRP,),
        in_specs=[pl.BlockSpec(memory_space=pl.ANY)],
        out_specs=[
            pl.BlockSpec((BGRP, K, 1), lambda i: (i, 0, 0)),
            pl.BlockSpec((BGRP, K, 1), lambda i: (i, 0, 0)),
            pl.BlockSpec((BGRP, K, 1), lambda i: (i, 0, 0)),
        ],
        out_shape=[
            jax.ShapeDtypeStruct((B, K, 1), jnp.int32),
            jax.ShapeDtypeStruct((B, K, 1), jnp.int32),
            jax.ShapeDtypeStruct((B, K, 1), jnp.float32),
        ],
        scratch_shapes=scratches,
    )(pred_logits)

    boxes = pl.pallas_call(
        _decode_kernel,
        grid=(B,),
        in_specs=[
            pl.BlockSpec((1, K, 1), lambda b: (b, 0, 0)),
            pl.BlockSpec((1, N, 4), lambda b: (b, 0, 0)),
            pl.BlockSpec((1, 1, 4), lambda b: (b, 0, 0)),
        ],
        out_specs=pl.BlockSpec((1, K, 4), lambda b: (b, 0, 0)),
        out_shape=jax.ShapeDtypeStruct((B, K, 4), jnp.float32),
    )(qidx, pred_boxes, scale4)

    return labels[:, :, 0], boxes, scores[:, :, 0]
